# baseline (speedup 1.0000x reference)
import functools

import jax
import jax.numpy as jnp
from jax import lax
from jax.experimental import pallas as pl
from jax.experimental.pallas import tpu as pltpu
from jax.experimental.pallas import tpu_sc as plsc

D = 64
CHUNK = 128


@functools.cache
def _build(N: int):
    info = plsc.get_sparse_core_info()
    NC, NS = info.num_cores, info.num_subcores
    NW = NC * NS
    n_per_w = N // NW
    n_chunks = n_per_w // CHUNK
    mesh = plsc.VectorSubcoreMesh(core_axis_name="c", subcore_axis_name="s")

    @functools.partial(
        pl.kernel,
        mesh=mesh,
        out_type=jax.ShapeDtypeStruct((N, D), jnp.float32),
        scratch_types=[
            pltpu.VMEM((n_chunks, CHUNK), jnp.int32),
            pltpu.VMEM((CHUNK, D), jnp.float32),
            pltpu.SemaphoreType.DMA,
        ],
        compiler_params=pltpu.CompilerParams(use_tc_tiling_on_sc=False),
    )
    def emb(idx_hbm, table_hbm, out_hbm, idx_v, rows_v, gsem):
        wid = lax.axis_index("s") * NC + lax.axis_index("c")
        base = wid * n_per_w
        pltpu.sync_copy(idx_hbm.at[wid], idx_v)

        def body(j, carry):
            pltpu.async_copy(table_hbm.at[idx_v.at[j]], rows_v, gsem).wait()
            pltpu.sync_copy(rows_v, out_hbm.at[pl.ds(base + j * CHUNK, CHUNK)])
            return carry

        lax.fori_loop(0, n_chunks, body, 0)

    return emb


def kernel(x, table):
    B, L = x.shape
    N = B * L
    info = plsc.get_sparse_core_info()
    NW = info.num_cores * info.num_subcores
    idx = x.astype(jnp.int32).reshape(NW, (N // NW) // CHUNK, CHUNK)
    out = _build(N)(idx, table)
    return out.reshape(B, L, D)


# speedup vs baseline: 1.0448x; 1.0448x over previous
import functools

import jax
import jax.numpy as jnp
from jax import lax
from jax.experimental import pallas as pl
from jax.experimental.pallas import tpu as pltpu
from jax.experimental.pallas import tpu_sc as plsc

D = 64
CHUNK = 128
NBUF = 5


@functools.cache
def _build(N: int):
    info = plsc.get_sparse_core_info()
    NC, NS = info.num_cores, info.num_subcores
    NW = NC * NS
    n_per_w = N // NW
    n_chunks = n_per_w // CHUNK
    assert n_chunks % NBUF == 0
    mesh = plsc.VectorSubcoreMesh(core_axis_name="c", subcore_axis_name="s")

    @functools.partial(
        pl.kernel,
        mesh=mesh,
        out_type=jax.ShapeDtypeStruct((N, D), jnp.float32),
        scratch_types=[
            pltpu.VMEM((n_chunks, CHUNK), jnp.int32),
            pltpu.VMEM((NBUF, CHUNK, D), jnp.float32),
        ]
        + [pltpu.SemaphoreType.DMA] * (2 * NBUF),
        compiler_params=pltpu.CompilerParams(use_tc_tiling_on_sc=False),
    )
    def emb(idx_hbm, table_hbm, out_hbm, idx_v, rows_v, *sems):
        gsem, ssem = sems[:NBUF], sems[NBUF:]
        wid = lax.axis_index("s") * NC + lax.axis_index("c")
        base = wid * n_per_w
        pltpu.sync_copy(idx_hbm.at[wid], idx_v)

        for s in range(NBUF):
            pltpu.async_copy(
                table_hbm.at[idx_v.at[s]], rows_v.at[s], gsem[s])

        def body(i, carry):
            for s in range(NBUF):
                j = i * NBUF + s
                pltpu.make_async_copy(
                    table_hbm.at[idx_v.at[j]], rows_v.at[s], gsem[s]).wait()
                dst = out_hbm.at[pl.ds(base + j * CHUNK, CHUNK)]
                pltpu.async_copy(rows_v.at[s], dst, ssem[s])
                pltpu.make_async_copy(rows_v.at[s], dst, ssem[s]).wait()

                @pl.when(j + NBUF < n_chunks)
                def _():
                    pltpu.async_copy(
                        table_hbm.at[idx_v.at[j + NBUF]],
                        rows_v.at[s], gsem[s])

            return carry

        lax.fori_loop(0, n_chunks // NBUF, body, 0)

    return emb


def kernel(x, table):
    B, L = x.shape
    N = B * L
    info = plsc.get_sparse_core_info()
    NW = info.num_cores * info.num_subcores
    idx = x.astype(jnp.int32).reshape(NW, (N // NW) // CHUNK, CHUNK)
    out = _build(N)(idx, table)
    return out.reshape(B, L, D)


# speedup vs baseline: 1.1920x; 1.1409x over previous
import functools

import jax
import jax.numpy as jnp
from jax import lax
from jax.experimental import pallas as pl
from jax.experimental.pallas import tpu as pltpu
from jax.experimental.pallas import tpu_sc as plsc

B = 4096
L = 50
D = 64
TOK = 128
NBLK = B * L // TOK


@functools.cache
def _build():
    info = plsc.get_sparse_core_info()
    NC, NS = info.num_cores, info.num_subcores
    NW = NC * NS
    n_per_w = B * L // NW
    g_per_w = n_per_w // TOK
    mesh = plsc.VectorSubcoreMesh(core_axis_name="c", subcore_axis_name="s")

    @functools.partial(
        pl.kernel,
        mesh=mesh,
        out_type=jax.ShapeDtypeStruct((L, D, B), jnp.float32),
        scratch_types=[
            pltpu.VMEM((g_per_w, TOK), jnp.int32),
            pltpu.VMEM((TOK, D), jnp.float32),
            pltpu.VMEM((TOK, D), jnp.float32),
            pltpu.VMEM((D, TOK), jnp.float32),
            pltpu.VMEM((D, TOK), jnp.float32),
        ]
        + [pltpu.SemaphoreType.DMA] * 4,
        compiler_params=pltpu.CompilerParams(
            use_tc_tiling_on_sc=True, needs_layout_passes=False),
    )
    def emb(idx_hbm, tab_hbm, out_hbm, idx_v, rows0, rows1, blk0, blk1,
            gsem0, gsem1, ssem0, ssem1):
        wid = lax.axis_index("s") * NC + lax.axis_index("c")
        t0w = wid * n_per_w
        pltpu.sync_copy(idx_hbm.at[wid], idx_v)
        lanes = lax.iota(jnp.int32, 16)
        lanesjg = [lanes + 16 * jg for jg in range(8)]

        def issue_fetch(g, rows, gsem):
            def fbody(jg, c):
                v = idx_v[g, pl.ds(jg * 16, 16)]
                for k in range(16):
                    pltpu.async_copy(
                        tab_hbm.at[pl.ds(v[k], 1)],
                        rows.at[pl.ds(jg * 16 + k, 1)], gsem)
                return c
            lax.fori_loop(0, 8, fbody, 0)

        def drain_fetch(rows, gsem):
            pltpu.make_async_copy(
                tab_hbm.at[pl.ds(0, TOK)], rows, gsem).wait()

        def transpose(rows, blk):
            def tbody(d, c):
                dv = lax.broadcast(d, (16,))
                for jg in range(8):
                    blk[d, pl.ds(jg * 16, 16)] = plsc.load_gather(
                        rows, [lanesjg[jg], dv])
                return c
            lax.fori_loop(0, D, tbody, 0)

        def store(g, blk, ssem):
            t0 = t0w + g * TOK
            l = t0 >> 12
            b0 = pl.multiple_of(t0 & (B - 1), TOK)
            pltpu.async_copy(
                blk, out_hbm.at[l, pl.ds(0, D), pl.ds(b0, TOK)], ssem)

        def drain_store(g, blk, ssem):
            t0 = t0w + g * TOK
            l = t0 >> 12
            b0 = pl.multiple_of(t0 & (B - 1), TOK)
            pltpu.make_async_copy(
                blk, out_hbm.at[l, pl.ds(0, D), pl.ds(b0, TOK)], ssem).wait()

        issue_fetch(0, rows0, gsem0)

        def pair(i, c):
            g0 = 2 * i
            drain_fetch(rows0, gsem0)
            issue_fetch(g0 + 1, rows1, gsem1)

            @pl.when(i > 0)
            def _():
                drain_store(g0 - 2, blk0, ssem0)

            transpose(rows0, blk0)
            store(g0, blk0, ssem0)

            drain_fetch(rows1, gsem1)

            @pl.when(g0 + 2 < g_per_w)
            def _():
                issue_fetch(g0 + 2, rows0, gsem0)

            @pl.when(i > 0)
            def _():
                drain_store(g0 - 1, blk1, ssem1)

            transpose(rows1, blk1)
            store(g0 + 1, blk1, ssem1)
            return c

        lax.fori_loop(0, g_per_w // 2, pair, 0)
        drain_store(g_per_w - 2, blk0, ssem0)
        drain_store(g_per_w - 1, blk1, ssem1)

    return emb


def kernel(x, table):
    info = plsc.get_sparse_core_info()
    NW = info.num_cores * info.num_subcores
    idx = x.astype(jnp.int32).T.reshape(NW, B * L // NW // TOK, TOK)
    out = _build()(idx, table)
    return out.transpose(2, 0, 1)
